# baseline (speedup 1.0000x reference)
import functools

import jax
import jax.numpy as jnp
from jax import lax
from jax.experimental import pallas as pl
from jax.experimental.pallas import tpu as pltpu
from jax.experimental.pallas import tpu_sc as plsc

BATCH = 16384
EMB = 64
NC = 2
NS = 16
NW = NC * NS
BPW = BATCH // NW
LANES = 16


def _body(p_hbm, q_hbm, uid_hbm, bid_hbm, out_hbm,
          uidx_v, bidx_v, urows_v, qrows_v, out_v, sem_u, sem_q):
    wid = lax.axis_index("s") * NC + lax.axis_index("c")
    base = wid * BPW

    pltpu.sync_copy(uid_hbm.at[pl.ds(base, BPW)], uidx_v)
    pltpu.sync_copy(bid_hbm.at[pl.ds(base, BPW)], bidx_v)

    cp_u = pltpu.async_copy(p_hbm.at[uidx_v], urows_v, sem_u)
    cp_q = pltpu.async_copy(q_hbm.at[bidx_v], qrows_v, sem_q)
    cp_u.wait()
    cp_q.wait()

    lanes = lax.iota(jnp.int32, LANES)

    def group(g, _):
        vec = jnp.zeros((LANES,), jnp.float32)
        for j in range(LANES):
            r = g * LANES + j
            acc = urows_v[r, pl.ds(0, LANES)] * qrows_v[r, pl.ds(0, LANES)]
            for k in range(1, EMB // LANES):
                acc = acc + urows_v[r, pl.ds(k * LANES, LANES)] * \
                    qrows_v[r, pl.ds(k * LANES, LANES)]
            vec = jnp.where(lanes == j, jnp.sum(acc), vec)
        out_v[pl.ds(g * LANES, LANES)] = vec
        return 0

    lax.fori_loop(0, BPW // LANES, group, 0)

    pltpu.sync_copy(out_v, out_hbm.at[pl.ds(base, BPW)])


_sc_call = pl.kernel(
    _body,
    out_type=jax.ShapeDtypeStruct((BATCH,), jnp.float32),
    mesh=plsc.VectorSubcoreMesh(
        core_axis_name="c", subcore_axis_name="s",
        num_cores=NC, num_subcores=NS),
    scratch_types=[
        pltpu.VMEM((BPW,), jnp.int32),
        pltpu.VMEM((BPW,), jnp.int32),
        pltpu.VMEM((BPW, EMB), jnp.float32),
        pltpu.VMEM((BPW, EMB), jnp.float32),
        pltpu.VMEM((BPW,), jnp.float32),
        pltpu.SemaphoreType.DMA,
        pltpu.SemaphoreType.DMA,
    ],
    compiler_params=pltpu.CompilerParams(
        needs_layout_passes=False, use_tc_tiling_on_sc=False),
)


@jax.jit
def kernel(P, Q, user_id, book_id):
    return _sc_call(P, Q, user_id.astype(jnp.int32), book_id.astype(jnp.int32))
